# baseline (speedup 1.0000x reference)
import functools

import jax
import jax.numpy as jnp
from jax import lax
from jax.experimental import pallas as pl
from jax.experimental.pallas import tpu as pltpu
from jax.experimental.pallas import tpu_sc as plsc

VOCAB = 1000000
EMBED_DIM = 32
BATCH = 16384

_NUM_CORES = 2
_NUM_SUBCORES = 16
_NW = _NUM_CORES * _NUM_SUBCORES
_B_PER_W = BATCH // _NW
_SLICES_PER_W = 2 * _B_PER_W

_mesh = plsc.VectorSubcoreMesh(core_axis_name="c", subcore_axis_name="s")


@functools.partial(
    pl.kernel,
    mesh=_mesh,
    out_type=jax.ShapeDtypeStruct((2 * BATCH, 16), jnp.float32),
    scratch_types=[
        pltpu.VMEM((_B_PER_W,), jnp.int32),
        pltpu.VMEM((_SLICES_PER_W,), jnp.int32),
        pltpu.VMEM((_SLICES_PER_W, 16), jnp.float32),
        pltpu.SemaphoreType.DMA,
    ],
    compiler_params=pltpu.CompilerParams(use_tc_tiling_on_sc=False, needs_layout_passes=False),
)
def _gather_kernel(idx_hbm, table_hbm, out_hbm, idx_v, j_v, rows_v, sem):
    wid = lax.axis_index("s") * _NUM_CORES + lax.axis_index("c")
    base = wid * _B_PER_W
    pltpu.sync_copy(idx_hbm.at[pl.ds(base, _B_PER_W)], idx_v)

    lane = lax.iota(jnp.int32, 16)

    def body(t, carry):
        j = t * 16 + lane
        k = lax.shift_right_logical(j, 1)
        v = plsc.load_gather(idx_v, [k])
        j_v[pl.ds(t * 16, 16)] = 2 * v + (j & 1)
        return carry

    lax.fori_loop(0, _SLICES_PER_W // 16, body, 0)

    pltpu.async_copy(table_hbm.at[j_v], rows_v, sem).wait()
    pltpu.sync_copy(rows_v, out_hbm.at[pl.ds(wid * _SLICES_PER_W, _SLICES_PER_W)])


def kernel(in_tensor, table):
    idx = in_tensor.reshape(BATCH).astype(jnp.int32)
    table2 = table.reshape(2 * VOCAB, 16)
    out = _gather_kernel(idx, table2)
    return out.reshape(BATCH, EMBED_DIM)
